# Initial kernel scaffold; baseline (speedup 1.0000x reference)
import jax
import jax.numpy as jnp
from jax.experimental import pallas as pl


def kernel(x, edge_index, batch, embed, W1, b1, W2, b2, Wl, bl):
    raise NotImplementedError("write your pallas kernel here")


# speedup vs baseline: 11.5398x; 11.5398x over previous
import functools

import jax
import jax.numpy as jnp
from jax import lax
from jax.experimental import pallas as pl
from jax.experimental.pallas import tpu as pltpu
from jax.experimental.pallas import tpu_sc as plsc

N = 10000
E = 320000
D = 128
NG = 128
NCLS = 16

KE = 80
EPT = E // 32
NCH = EPT // KE
XCH = N // KE

_mesh = plsc.VectorSubcoreMesh(core_axis_name="c", subcore_axis_name="s")


def _rr16(s, nch, body):
    for k in range(nch // 16):
        body(s + 16 * k)
    rem = nch % 16
    if rem:
        @pl.when(s < rem)
        def _():
            body(s + 16 * (nch // 16))


@functools.partial(
    pl.kernel,
    out_type=(
        jax.ShapeDtypeStruct((2, N, D), jnp.float32),
        jax.ShapeDtypeStruct((N, D), jnp.float32),
    ),
    mesh=_mesh,
    scratch_types=[
        pltpu.VMEM_SHARED((N, D), jnp.float32),
        pltpu.VMEM((KE,), jnp.int32),
        pltpu.VMEM((KE, D), jnp.float32),
        pltpu.VMEM((KE, D), jnp.float32),
        pltpu.VMEM((KE,), jnp.int32),
        pltpu.VMEM((KE, D), jnp.float32),
        pltpu.SemaphoreType.DMA,
    ],
)
def _sc_deg_emb(col_hbm, x_hbm, embed_hbm, degp_hbm, h0_hbm,
                dacc, cidx, ones_v, zeros_v, xidx, rows, sem):
    c = lax.axis_index("c")
    s = lax.axis_index("s")
    w = s * 2 + c

    zv = jnp.zeros((16,), jnp.float32)
    ov = jnp.full((16,), 1.0, jnp.float32)

    def fill(i, _):
        for jj in range(D // 16):
            zeros_v[i, pl.ds(jj * 16, 16)] = zv
            ones_v[i, pl.ds(jj * 16, 16)] = ov
        return 0
    lax.fori_loop(0, KE, fill, 0)

    _rr16(s, XCH, lambda ch: pltpu.sync_copy(zeros_v, dacc.at[pl.ds(ch * KE, KE)]))
    plsc.subcore_barrier()

    ebase = (c * 16 + s) * EPT

    def deg_it(j, _):
        b = ebase + j * KE
        pltpu.sync_copy(col_hbm.at[pl.ds(b, KE)], cidx)
        pltpu.sync_copy(ones_v, dacc.at[cidx], add=True)
        return 0
    lax.fori_loop(0, NCH, deg_it, 0)

    def emb_chunk(chunk):
        b = chunk * KE
        pltpu.sync_copy(x_hbm.at[pl.ds(b, KE)], xidx)
        pltpu.async_copy(embed_hbm.at[xidx], rows, sem).wait()
        pltpu.sync_copy(rows, h0_hbm.at[pl.ds(b, KE)])

    for k in range(XCH // 32):
        emb_chunk(w + 32 * k)
    rem = XCH % 32
    if rem:
        @pl.when(w < rem)
        def _():
            emb_chunk(w + 32 * (XCH // 32))

    plsc.subcore_barrier()
    _rr16(s, XCH, lambda ch: pltpu.sync_copy(
        dacc.at[pl.ds(ch * KE, KE)], degp_hbm.at[c, pl.ds(ch * KE, KE)]))


@functools.partial(
    pl.kernel,
    out_type=jax.ShapeDtypeStruct((2, N, D), jnp.float32),
    mesh=_mesh,
    scratch_types=[
        pltpu.VMEM_SHARED((N, D), jnp.float32),
        pltpu.VMEM((KE,), jnp.int32),
        pltpu.VMEM((KE,), jnp.int32),
        pltpu.VMEM((KE, D), jnp.float32),
        pltpu.VMEM((KE, D), jnp.float32),
        pltpu.SemaphoreType.DMA,
    ],
)
def _sc_edge(row_hbm, col_hbm, g_hbm, accp_hbm,
             acc, ridx, cidx, rows, zeros_v, sem):
    c = lax.axis_index("c")
    s = lax.axis_index("s")

    zv = jnp.zeros((16,), jnp.float32)

    def fill(i, _):
        for jj in range(D // 16):
            zeros_v[i, pl.ds(jj * 16, 16)] = zv
        return 0
    lax.fori_loop(0, KE, fill, 0)

    _rr16(s, XCH, lambda ch: pltpu.sync_copy(zeros_v, acc.at[pl.ds(ch * KE, KE)]))
    plsc.subcore_barrier()

    ebase = (c * 16 + s) * EPT

    def edge_it(j, _):
        b = ebase + j * KE
        pltpu.sync_copy(row_hbm.at[pl.ds(b, KE)], ridx)
        pltpu.async_copy(g_hbm.at[ridx], rows, sem).wait()
        pltpu.sync_copy(col_hbm.at[pl.ds(b, KE)], cidx)
        pltpu.sync_copy(rows, acc.at[cidx], add=True)
        return 0
    lax.fori_loop(0, NCH, edge_it, 0)

    plsc.subcore_barrier()
    _rr16(s, XCH, lambda ch: pltpu.sync_copy(
        acc.at[pl.ds(ch * KE, KE)], accp_hbm.at[c, pl.ds(ch * KE, KE)]))


def _tc1_body(degp_ref, h0_ref, w1_ref, g1_ref, dis_ref):
    d16 = degp_ref[0] + degp_ref[1]
    dis = lax.rsqrt(d16[:, 0:1] + 1.0)
    g1_ref[...] = jnp.dot(h0_ref[...], w1_ref[...],
                          preferred_element_type=jnp.float32) * dis
    dis_ref[...] = dis


def _tc2_body(accp_ref, g1_ref, dis_ref, b1_ref, w2_ref, g2_ref):
    dis = dis_ref[...]
    h1 = jnp.maximum(dis * (accp_ref[0] + accp_ref[1] + g1_ref[...])
                     + b1_ref[...], 0.0)
    g2_ref[...] = jnp.dot(h1, w2_ref[...],
                          preferred_element_type=jnp.float32) * dis


def _tc3_body(accp_ref, g2_ref, dis_ref, b2_ref, batch_ref, wl_ref, bl_ref,
              out_ref):
    dis = dis_ref[...]
    h2 = jnp.maximum(dis * (accp_ref[0] + accp_ref[1] + g2_ref[...])
                     + b2_ref[...], 0.0)
    gid = lax.broadcasted_iota(jnp.int32, (N, NG), 1)
    eh = (batch_ref[...] == gid).astype(jnp.float32)
    sums = lax.dot_general(eh, h2, (((0,), (0,)), ((), ())),
                           preferred_element_type=jnp.float32)
    cnt = lax.dot_general(eh, jnp.ones((N, 1), jnp.float32),
                          (((0,), (0,)), ((), ())),
                          preferred_element_type=jnp.float32)
    pooled = sums / jnp.maximum(cnt, 1.0)
    out_ref[...] = jnp.dot(pooled, wl_ref[...],
                           preferred_element_type=jnp.float32) + bl_ref[...]


_tc1 = pl.pallas_call(
    _tc1_body,
    out_shape=(jax.ShapeDtypeStruct((N, D), jnp.float32),
               jax.ShapeDtypeStruct((N, 1), jnp.float32)))
_tc2 = pl.pallas_call(
    _tc2_body,
    out_shape=jax.ShapeDtypeStruct((N, D), jnp.float32))
_tc3 = pl.pallas_call(
    _tc3_body,
    out_shape=jax.ShapeDtypeStruct((NG, NCLS), jnp.float32))


def kernel(x, edge_index, batch, embed, W1, b1, W2, b2, Wl, bl):
    x = x.reshape(-1).astype(jnp.int32)
    row = edge_index[0].astype(jnp.int32)
    col = edge_index[1].astype(jnp.int32)
    batch2 = batch.reshape(-1, 1).astype(jnp.int32)

    degp, h0 = _sc_deg_emb(col, x, embed)
    g1, dis = _tc1(degp, h0, W1)
    acc1 = _sc_edge(row, col, g1)
    g2 = _tc2(acc1, g1, dis, b1.reshape(1, -1), W2)
    acc2 = _sc_edge(row, col, g2)
    return _tc3(acc2, g2, dis, b2.reshape(1, -1), batch2, Wl, bl.reshape(1, -1))


# speedup vs baseline: 25.2151x; 2.1851x over previous
import functools

import jax
import jax.numpy as jnp
from jax import lax
from jax.experimental import pallas as pl
from jax.experimental.pallas import tpu as pltpu
from jax.experimental.pallas import tpu_sc as plsc

N = 10000
E = 320000
D = 128
NG = 128
NCLS = 16

KE = 80
EPT = E // 32
NCH = EPT // KE
XCH = N // KE

_mesh = plsc.VectorSubcoreMesh(core_axis_name="c", subcore_axis_name="s")


def _rr16(s, nch, body):
    for k in range(nch // 16):
        body(s + 16 * k)
    rem = nch % 16
    if rem:
        @pl.when(s < rem)
        def _():
            body(s + 16 * (nch // 16))


@functools.partial(
    pl.kernel,
    out_type=(
        jax.ShapeDtypeStruct((2, N, D), jnp.float32),
        jax.ShapeDtypeStruct((N, D), jnp.float32),
    ),
    mesh=_mesh,
    scratch_types=[
        pltpu.VMEM_SHARED((N, D), jnp.float32),
        pltpu.VMEM((KE,), jnp.int32),
        pltpu.VMEM((KE,), jnp.int32),
        pltpu.VMEM((KE, D), jnp.float32),
        pltpu.VMEM((KE, D), jnp.float32),
        pltpu.VMEM((4 * KE,), jnp.int32),
        pltpu.VMEM((KE, D), jnp.float32),
        pltpu.VMEM((KE, D), jnp.float32),
        pltpu.SemaphoreType.DMA,
        pltpu.SemaphoreType.DMA,
        pltpu.SemaphoreType.DMA,
    ],
)
def _sc_deg_emb(col_hbm, x_hbm, embed_hbm, degp_hbm, h0_hbm,
                dacc, cidx0, cidx1, ones_v, zeros_v, xidx, rows0, rows1,
                semc0, semc1, seme):
    c = lax.axis_index("c")
    s = lax.axis_index("s")
    w = s * 2 + c

    zv = jnp.zeros((16,), jnp.float32)
    ov = jnp.full((16,), 1.0, jnp.float32)

    def fill(i, _):
        for jj in range(D // 16):
            zeros_v[i, pl.ds(jj * 16, 16)] = zv
            ones_v[i, pl.ds(jj * 16, 16)] = ov
        return 0
    lax.fori_loop(0, KE, fill, 0)

    erows = [rows0, rows1]
    nfull = XCH // 32
    rem = XCH % 32
    my_chunks = [w + 32 * k for k in range(nfull)]
    for k, ch in enumerate(my_chunks):
        pltpu.sync_copy(x_hbm.at[pl.ds(ch * KE, KE)],
                        xidx.at[pl.ds(k * KE, KE)])
    extra = w + 32 * nfull
    @pl.when(w < rem)
    def _():
        pltpu.sync_copy(x_hbm.at[pl.ds(extra * KE, KE)],
                        xidx.at[pl.ds(nfull * KE, KE)])

    def fire_emb(k, buf):
        return pltpu.async_copy(
            embed_hbm.at[xidx.at[pl.ds(k * KE, KE)]], erows[buf], seme)

    h0_ = fire_emb(0, 0)
    h1_ = fire_emb(1, 1)

    _rr16(s, XCH, lambda ch: pltpu.sync_copy(zeros_v, dacc.at[pl.ds(ch * KE, KE)]))
    plsc.subcore_barrier()

    ebase = (c * 16 + s) * EPT

    def start_d(j, cb, sc_):
        off = pl.multiple_of((ebase + j * KE), 8)
        pltpu.async_copy(col_hbm.at[pl.ds(off, KE)], cb, sc_)

    def finish_d(cb, sc_):
        pltpu.make_async_copy(col_hbm.at[pl.ds(0, KE)], cb, sc_).wait()
        pltpu.sync_copy(ones_v, dacc.at[cb], add=True)

    start_d(0, cidx0, semc0)

    def deg_it(k, _):
        start_d(2 * k + 1, cidx1, semc1)
        finish_d(cidx0, semc0)
        start_d(2 * k + 2, cidx0, semc0)
        finish_d(cidx1, semc1)
        return 0
    lax.fori_loop(0, (NCH - 1) // 2, deg_it, 0)
    finish_d(cidx0, semc0)

    h0_.wait()
    pltpu.sync_copy(erows[0], h0_hbm.at[pl.ds(my_chunks[0] * KE, KE)])
    h2_ = fire_emb(2, 0)
    h1_.wait()
    pltpu.sync_copy(erows[1], h0_hbm.at[pl.ds(my_chunks[1] * KE, KE)])
    @pl.when(w < rem)
    def _():
        fire_emb(nfull, 1)
    h2_.wait()
    pltpu.sync_copy(erows[0], h0_hbm.at[pl.ds(my_chunks[2] * KE, KE)])
    @pl.when(w < rem)
    def _():
        pltpu.make_async_copy(embed_hbm.at[xidx.at[pl.ds(nfull * KE, KE)]],
                              erows[1], seme).wait()
        pltpu.sync_copy(erows[1], h0_hbm.at[pl.ds(extra * KE, KE)])

    plsc.subcore_barrier()
    _rr16(s, XCH, lambda ch: pltpu.sync_copy(
        dacc.at[pl.ds(ch * KE, KE)], degp_hbm.at[c, pl.ds(ch * KE, KE)]))


@functools.partial(
    pl.kernel,
    out_type=jax.ShapeDtypeStruct((2, N, D), jnp.float32),
    mesh=_mesh,
    scratch_types=[
        pltpu.VMEM_SHARED((N, D), jnp.float32),
        pltpu.VMEM((EPT,), jnp.int32),
        pltpu.VMEM((KE,), jnp.int32),
        pltpu.VMEM((KE,), jnp.int32),
        pltpu.VMEM((KE, D), jnp.float32),
        pltpu.VMEM((KE, D), jnp.float32),
        pltpu.VMEM((KE, D), jnp.float32),
        pltpu.SemaphoreType.DMA,
        pltpu.SemaphoreType.DMA,
        pltpu.SemaphoreType.DMA,
        pltpu.SemaphoreType.DMA,
    ],
)
def _sc_edge(row_hbm, col_hbm, g_hbm, accp_hbm,
             acc, ridx_all, cidx0, cidx1, rows0, rows1, zeros_v,
             semg0, semg1, semc0, semc1):
    c = lax.axis_index("c")
    s = lax.axis_index("s")

    zv = jnp.zeros((16,), jnp.float32)

    def fill(i, _):
        for jj in range(D // 16):
            zeros_v[i, pl.ds(jj * 16, 16)] = zv
        return 0
    lax.fori_loop(0, KE, fill, 0)

    ebase = (c * 16 + s) * EPT
    pltpu.sync_copy(row_hbm.at[pl.ds(ebase, EPT)], ridx_all)

    _rr16(s, XCH, lambda ch: pltpu.sync_copy(zeros_v, acc.at[pl.ds(ch * KE, KE)]))
    plsc.subcore_barrier()

    def start_e(j, cb, rb, sc_, sg):
        off = pl.multiple_of(j * KE, 8)
        pltpu.async_copy(col_hbm.at[pl.ds(ebase + off, KE)], cb, sc_)
        pltpu.async_copy(g_hbm.at[ridx_all.at[pl.ds(off, KE)]], rb, sg)

    def finish_e(cb, rb, sc_, sg):
        pltpu.make_async_copy(g_hbm.at[ridx_all.at[pl.ds(0, KE)]], rb, sg).wait()
        pltpu.make_async_copy(col_hbm.at[pl.ds(0, KE)], cb, sc_).wait()
        pltpu.sync_copy(rb, acc.at[cb], add=True)

    start_e(0, cidx0, rows0, semc0, semg0)

    def edge_it(k, _):
        start_e(2 * k + 1, cidx1, rows1, semc1, semg1)
        finish_e(cidx0, rows0, semc0, semg0)
        start_e(2 * k + 2, cidx0, rows0, semc0, semg0)
        finish_e(cidx1, rows1, semc1, semg1)
        return 0
    lax.fori_loop(0, (NCH - 1) // 2, edge_it, 0)
    finish_e(cidx0, rows0, semc0, semg0)

    plsc.subcore_barrier()
    _rr16(s, XCH, lambda ch: pltpu.sync_copy(
        acc.at[pl.ds(ch * KE, KE)], accp_hbm.at[c, pl.ds(ch * KE, KE)]))


def _tc1_body(degp_ref, h0_ref, w1_ref, g1_ref, dis_ref):
    d16 = degp_ref[0] + degp_ref[1]
    dis = lax.rsqrt(d16[:, 0:1] + 1.0)
    g1_ref[...] = jnp.dot(h0_ref[...], w1_ref[...],
                          preferred_element_type=jnp.float32) * dis
    dis_ref[...] = dis


def _tc2_body(accp_ref, g1_ref, dis_ref, b1_ref, w2_ref, g2_ref):
    dis = dis_ref[...]
    h1 = jnp.maximum(dis * (accp_ref[0] + accp_ref[1] + g1_ref[...])
                     + b1_ref[...], 0.0)
    g2_ref[...] = jnp.dot(h1, w2_ref[...],
                          preferred_element_type=jnp.float32) * dis


def _tc3_body(accp_ref, g2_ref, dis_ref, b2_ref, batch_ref, wl_ref, bl_ref,
              out_ref):
    dis = dis_ref[...]
    h2 = jnp.maximum(dis * (accp_ref[0] + accp_ref[1] + g2_ref[...])
                     + b2_ref[...], 0.0)
    gid = lax.broadcasted_iota(jnp.int32, (N, NG), 1)
    eh = (batch_ref[...] == gid).astype(jnp.float32)
    sums = lax.dot_general(eh, h2, (((0,), (0,)), ((), ())),
                           preferred_element_type=jnp.float32)
    cnt = lax.dot_general(eh, jnp.ones((N, 1), jnp.float32),
                          (((0,), (0,)), ((), ())),
                          preferred_element_type=jnp.float32)
    pooled = sums / jnp.maximum(cnt, 1.0)
    out_ref[...] = jnp.dot(pooled, wl_ref[...],
                           preferred_element_type=jnp.float32) + bl_ref[...]


_tc1 = pl.pallas_call(
    _tc1_body,
    out_shape=(jax.ShapeDtypeStruct((N, D), jnp.float32),
               jax.ShapeDtypeStruct((N, 1), jnp.float32)))
_tc2 = pl.pallas_call(
    _tc2_body,
    out_shape=jax.ShapeDtypeStruct((N, D), jnp.float32))
_tc3 = pl.pallas_call(
    _tc3_body,
    out_shape=jax.ShapeDtypeStruct((NG, NCLS), jnp.float32))


def kernel(x, edge_index, batch, embed, W1, b1, W2, b2, Wl, bl):
    x = x.reshape(-1).astype(jnp.int32)
    row = edge_index[0].astype(jnp.int32)
    col = edge_index[1].astype(jnp.int32)
    batch2 = batch.reshape(-1, 1).astype(jnp.int32)

    degp, h0 = _sc_deg_emb(col, x, embed)
    g1, dis = _tc1(degp, h0, W1)
    acc1 = _sc_edge(row, col, g1)
    g2 = _tc2(acc1, g1, dis, b1.reshape(1, -1), W2)
    acc2 = _sc_edge(row, col, g2)
    return _tc3(acc2, g2, dis, b2.reshape(1, -1), batch2, Wl, bl.reshape(1, -1))


# speedup vs baseline: 28.3130x; 1.1229x over previous
import functools

import jax
import jax.numpy as jnp
from jax import lax
from jax.experimental import pallas as pl
from jax.experimental.pallas import tpu as pltpu
from jax.experimental.pallas import tpu_sc as plsc

N = 10000
E = 320000
D = 128
NG = 128
NCLS = 16

KE = 80
EPT = E // 32
NCH = EPT // KE
XCH = N // KE

_mesh = plsc.VectorSubcoreMesh(core_axis_name="c", subcore_axis_name="s")


def _rr16(s, nch, body):
    for k in range(nch // 16):
        body(s + 16 * k)
    rem = nch % 16
    if rem:
        @pl.when(s < rem)
        def _():
            body(s + 16 * (nch // 16))


@functools.partial(
    pl.kernel,
    out_type=(
        jax.ShapeDtypeStruct((2, N, D), jnp.float32),
        jax.ShapeDtypeStruct((N, D), jnp.float32),
    ),
    mesh=_mesh,
    scratch_types=[
        pltpu.VMEM_SHARED((N, D), jnp.float32),
        pltpu.VMEM((KE,), jnp.int32),
        pltpu.VMEM((KE,), jnp.int32),
        pltpu.VMEM((KE,), jnp.int32),
        pltpu.VMEM((KE,), jnp.int32),
        pltpu.VMEM((KE, D), jnp.float32),
        pltpu.VMEM((KE, D), jnp.float32),
        pltpu.VMEM((4 * KE,), jnp.int32),
        pltpu.VMEM((KE, D), jnp.float32),
        pltpu.VMEM((KE, D), jnp.float32),
        pltpu.SemaphoreType.DMA,
        pltpu.SemaphoreType.DMA,
        pltpu.SemaphoreType.DMA,
        pltpu.SemaphoreType.DMA,
        pltpu.SemaphoreType.DMA,
        pltpu.SemaphoreType.DMA,
        pltpu.SemaphoreType.DMA,
        pltpu.SemaphoreType.DMA,
        pltpu.SemaphoreType.DMA,
    ],
)
def _sc_deg_emb(col_hbm, x_hbm, embed_hbm, degp_hbm, h0_hbm,
                dacc, ci0, ci1, ci2, ci3, ones_v, zeros_v, xidx, rows0, rows1,
                sc0, sc1, sc2, sc3, ss0, ss1, ss2, ss3, seme):
    c = lax.axis_index("c")
    s = lax.axis_index("s")
    w = s * 2 + c
    cidx = [ci0, ci1, ci2, ci3]
    semc = [sc0, sc1, sc2, sc3]
    sems = [ss0, ss1, ss2, ss3]

    zv = jnp.zeros((16,), jnp.float32)
    ov = jnp.full((16,), 1.0, jnp.float32)

    def fill(i, _):
        for jj in range(D // 16):
            zeros_v[i, pl.ds(jj * 16, 16)] = zv
            ones_v[i, pl.ds(jj * 16, 16)] = ov
        return 0
    lax.fori_loop(0, KE, fill, 0)

    erows = [rows0, rows1]
    nfull = XCH // 32
    rem = XCH % 32
    my_chunks = [w + 32 * k for k in range(nfull)]
    for k, ch in enumerate(my_chunks):
        pltpu.sync_copy(x_hbm.at[pl.ds(ch * KE, KE)],
                        xidx.at[pl.ds(k * KE, KE)])
    extra = w + 32 * nfull
    @pl.when(w < rem)
    def _():
        pltpu.sync_copy(x_hbm.at[pl.ds(extra * KE, KE)],
                        xidx.at[pl.ds(nfull * KE, KE)])

    def fire_emb(k, buf):
        return pltpu.async_copy(
            embed_hbm.at[xidx.at[pl.ds(k * KE, KE)]], erows[buf], seme)

    h0_ = fire_emb(0, 0)
    h1_ = fire_emb(1, 1)

    ebase = (c * 16 + s) * EPT

    def load_c(j, b):
        off = pl.multiple_of(ebase + j * KE, 8)
        pltpu.async_copy(col_hbm.at[pl.ds(off, KE)], cidx[b], semc[b])

    def wait_c(b):
        pltpu.make_async_copy(col_hbm.at[pl.ds(0, KE)], cidx[b], semc[b]).wait()

    def scat_d(b):
        pltpu.async_copy(ones_v, dacc.at[cidx[b]], sems[b], add=True)

    def wait_s(b):
        pltpu.make_async_copy(ones_v, dacc.at[cidx[b]], sems[b]).wait()

    load_c(0, 0)
    load_c(1, 1)
    load_c(2, 2)

    _rr16(s, XCH, lambda ch: pltpu.sync_copy(zeros_v, dacc.at[pl.ds(ch * KE, KE)]))
    plsc.subcore_barrier()

    wait_c(0)
    scat_d(0)
    load_c(3, 3)
    wait_c(1)
    scat_d(1)
    wait_c(2)
    scat_d(2)

    def deg_it(k, _):
        for i in range(4):
            j = 4 * k + 3 + i
            b = (3 + i) % 4
            wait_c(b)
            scat_d(b)
            wait_s((b + 1) % 4)
            load_c(j + 1, (b + 1) % 4)
        return 0
    lax.fori_loop(0, 30, deg_it, 0)

    wait_c(3)
    scat_d(3)
    wait_s(0)
    load_c(124, 0)
    wait_c(0)
    scat_d(0)
    wait_s(1)
    wait_s(2)
    wait_s(3)
    wait_s(0)

    h0_.wait()
    pltpu.sync_copy(erows[0], h0_hbm.at[pl.ds(my_chunks[0] * KE, KE)])
    h2_ = fire_emb(2, 0)
    h1_.wait()
    pltpu.sync_copy(erows[1], h0_hbm.at[pl.ds(my_chunks[1] * KE, KE)])
    @pl.when(w < rem)
    def _():
        fire_emb(nfull, 1)
    h2_.wait()
    pltpu.sync_copy(erows[0], h0_hbm.at[pl.ds(my_chunks[2] * KE, KE)])
    @pl.when(w < rem)
    def _():
        pltpu.make_async_copy(embed_hbm.at[xidx.at[pl.ds(nfull * KE, KE)]],
                              erows[1], seme).wait()
        pltpu.sync_copy(erows[1], h0_hbm.at[pl.ds(extra * KE, KE)])

    plsc.subcore_barrier()
    _rr16(s, XCH, lambda ch: pltpu.sync_copy(
        dacc.at[pl.ds(ch * KE, KE)], degp_hbm.at[c, pl.ds(ch * KE, KE)]))


@functools.partial(
    pl.kernel,
    out_type=jax.ShapeDtypeStruct((2, N, D), jnp.float32),
    mesh=_mesh,
    scratch_types=[
        pltpu.VMEM_SHARED((N, D), jnp.float32),
        pltpu.VMEM((KE,), jnp.int32),
        pltpu.VMEM((KE,), jnp.int32),
        pltpu.VMEM((KE,), jnp.int32),
        pltpu.VMEM((KE,), jnp.int32),
        pltpu.VMEM((KE,), jnp.int32),
        pltpu.VMEM((KE,), jnp.int32),
        pltpu.VMEM((KE,), jnp.int32),
        pltpu.VMEM((KE,), jnp.int32),
        pltpu.VMEM((KE, D), jnp.float32),
        pltpu.VMEM((KE, D), jnp.float32),
        pltpu.VMEM((KE, D), jnp.float32),
        pltpu.VMEM((KE, D), jnp.float32),
        pltpu.SemaphoreType.DMA,
        pltpu.SemaphoreType.DMA,
        pltpu.SemaphoreType.DMA,
        pltpu.SemaphoreType.DMA,
        pltpu.SemaphoreType.DMA,
        pltpu.SemaphoreType.DMA,
        pltpu.SemaphoreType.DMA,
        pltpu.SemaphoreType.DMA,
        pltpu.SemaphoreType.DMA,
        pltpu.SemaphoreType.DMA,
        pltpu.SemaphoreType.DMA,
        pltpu.SemaphoreType.DMA,
    ],
)
def _sc_edge(row_hbm, col_hbm, g_hbm, accp_hbm,
             acc, ri0, ri1, ri2, ri3, ci0, ci1, ci2, ci3,
             rw0, rw1, rw2, rw3,
             si0, si1, si2, si3, sg0, sg1, sg2, sg3, ss0, ss1, ss2, ss3):
    c = lax.axis_index("c")
    s = lax.axis_index("s")
    ridx = [ri0, ri1, ri2, ri3]
    cidx = [ci0, ci1, ci2, ci3]
    rows = [rw0, rw1, rw2, rw3]
    semi = [si0, si1, si2, si3]
    semg = [sg0, sg1, sg2, sg3]
    sems = [ss0, ss1, ss2, ss3]

    zv = jnp.zeros((16,), jnp.float32)

    def fill(i, _):
        for jj in range(D // 16):
            rw0[i, pl.ds(jj * 16, 16)] = zv
        return 0
    lax.fori_loop(0, KE, fill, 0)

    ebase = (c * 16 + s) * EPT

    def load_idx(j, b):
        off = pl.multiple_of(ebase + j * KE, 8)
        pltpu.async_copy(row_hbm.at[pl.ds(off, KE)], ridx[b], semi[b])
        pltpu.async_copy(col_hbm.at[pl.ds(off, KE)], cidx[b], semi[b])

    def wait_idx(b):
        pltpu.make_async_copy(row_hbm.at[pl.ds(0, KE)], ridx[b], semi[b]).wait()
        pltpu.make_async_copy(col_hbm.at[pl.ds(0, KE)], cidx[b], semi[b]).wait()

    def gather(b):
        pltpu.async_copy(g_hbm.at[ridx[b]], rows[b], semg[b])

    def wait_gather(b):
        pltpu.make_async_copy(g_hbm.at[ridx[b]], rows[b], semg[b]).wait()

    def scatter(b):
        pltpu.async_copy(rows[b], acc.at[cidx[b]], sems[b], add=True)

    def wait_scatter(b):
        pltpu.make_async_copy(rows[b], acc.at[cidx[b]], sems[b]).wait()

    load_idx(0, 0)
    load_idx(1, 1)
    load_idx(2, 2)
    _rr16(s, XCH, lambda ch: pltpu.sync_copy(rw0, acc.at[pl.ds(ch * KE, KE)]))
    plsc.subcore_barrier()

    wait_idx(0)
    gather(0)
    wait_idx(1)
    gather(1)

    wait_gather(0)
    scatter(0)
    load_idx(3, 3)
    wait_idx(2)
    gather(2)

    def edge_it(k, _):
        for i in range(4):
            j = 4 * k + 1 + i
            b = (1 + i) % 4
            wait_gather(b)
            scatter(b)
            wait_scatter((b + 3) % 4)
            load_idx(j + 3, (b + 3) % 4)
            wait_idx((b + 2) % 4)
            gather((b + 2) % 4)
        return 0
    lax.fori_loop(0, 30, edge_it, 0)

    wait_gather(1)
    scatter(1)
    wait_scatter(0)
    load_idx(124, 0)
    wait_idx(3)
    gather(3)
    wait_gather(2)
    scatter(2)
    wait_idx(0)
    gather(0)
    wait_gather(3)
    scatter(3)
    wait_gather(0)
    scatter(0)
    wait_scatter(1)
    wait_scatter(2)
    wait_scatter(3)
    wait_scatter(0)

    plsc.subcore_barrier()
    _rr16(s, XCH, lambda ch: pltpu.sync_copy(
        acc.at[pl.ds(ch * KE, KE)], accp_hbm.at[c, pl.ds(ch * KE, KE)]))


def _tc1_body(degp_ref, h0_ref, w1_ref, g1_ref, dis_ref):
    d16 = degp_ref[0] + degp_ref[1]
    dis = lax.rsqrt(d16[:, 0:1] + 1.0)
    g1_ref[...] = jnp.dot(h0_ref[...], w1_ref[...],
                          preferred_element_type=jnp.float32) * dis
    dis_ref[...] = dis


def _tc2_body(accp_ref, g1_ref, dis_ref, b1_ref, w2_ref, g2_ref):
    dis = dis_ref[...]
    h1 = jnp.maximum(dis * (accp_ref[0] + accp_ref[1] + g1_ref[...])
                     + b1_ref[...], 0.0)
    g2_ref[...] = jnp.dot(h1, w2_ref[...],
                          preferred_element_type=jnp.float32) * dis


def _tc3_body(accp_ref, g2_ref, dis_ref, b2_ref, batch_ref, wl_ref, bl_ref,
              out_ref):
    dis = dis_ref[...]
    h2 = jnp.maximum(dis * (accp_ref[0] + accp_ref[1] + g2_ref[...])
                     + b2_ref[...], 0.0)
    gid = lax.broadcasted_iota(jnp.int32, (N, NG), 1)
    eh = (batch_ref[...] == gid).astype(jnp.float32)
    sums = lax.dot_general(eh, h2, (((0,), (0,)), ((), ())),
                           preferred_element_type=jnp.float32)
    cnt = lax.dot_general(eh, jnp.ones((N, 1), jnp.float32),
                          (((0,), (0,)), ((), ())),
                          preferred_element_type=jnp.float32)
    pooled = sums / jnp.maximum(cnt, 1.0)
    out_ref[...] = jnp.dot(pooled, wl_ref[...],
                           preferred_element_type=jnp.float32) + bl_ref[...]


_tc1 = pl.pallas_call(
    _tc1_body,
    out_shape=(jax.ShapeDtypeStruct((N, D), jnp.float32),
               jax.ShapeDtypeStruct((N, 1), jnp.float32)))
_tc2 = pl.pallas_call(
    _tc2_body,
    out_shape=jax.ShapeDtypeStruct((N, D), jnp.float32))
_tc3 = pl.pallas_call(
    _tc3_body,
    out_shape=jax.ShapeDtypeStruct((NG, NCLS), jnp.float32))


def kernel(x, edge_index, batch, embed, W1, b1, W2, b2, Wl, bl):
    x = x.reshape(-1).astype(jnp.int32)
    row = edge_index[0].astype(jnp.int32)
    col = edge_index[1].astype(jnp.int32)
    batch2 = batch.reshape(-1, 1).astype(jnp.int32)

    degp, h0 = _sc_deg_emb(col, x, embed)
    g1, dis = _tc1(degp, h0, W1)
    acc1 = _sc_edge(row, col, g1)
    g2 = _tc2(acc1, g1, dis, b1.reshape(1, -1), W2)
    acc2 = _sc_edge(row, col, g2)
    return _tc3(acc2, g2, dis, b2.reshape(1, -1), batch2, Wl, bl.reshape(1, -1))


# speedup vs baseline: 30.8725x; 1.0904x over previous
import functools

import jax
import jax.numpy as jnp
from jax import lax
from jax.experimental import pallas as pl
from jax.experimental.pallas import tpu as pltpu
from jax.experimental.pallas import tpu_sc as plsc

N = 10000
E = 320000
D = 128
NG = 128
NCLS = 16

KE = 80
EPT = E // 32
NCH = EPT // KE
XCH = N // KE

_mesh = plsc.VectorSubcoreMesh(core_axis_name="c", subcore_axis_name="s")


def _rr16(s, nch, body):
    for k in range(nch // 16):
        body(s + 16 * k)
    rem = nch % 16
    if rem:
        @pl.when(s < rem)
        def _():
            body(s + 16 * (nch // 16))


@functools.partial(
    pl.kernel,
    out_type=(
        jax.ShapeDtypeStruct((2, 2, 16, 16, N // 2), jnp.float32),
        jax.ShapeDtypeStruct((N, D), jnp.float32),
    ),
    mesh=_mesh,
    scratch_types=[
        pltpu.VMEM((EPT,), jnp.int32),
        pltpu.VMEM((16, N // 2), jnp.float32),
        pltpu.VMEM((4 * KE,), jnp.int32),
        pltpu.VMEM((KE, D), jnp.float32),
        pltpu.VMEM((KE, D), jnp.float32),
        pltpu.SemaphoreType.DMA,
        pltpu.SemaphoreType.DMA,
    ],
    compiler_params=pltpu.CompilerParams(needs_layout_passes=False),
)
def _sc_deg_emb(col_hbm, x_hbm, embed_hbm, degp_hbm, h0_hbm,
                colbuf, hist, xidx, rows0, rows1, semc, seme):
    c = lax.axis_index("c")
    s = lax.axis_index("s")
    w = s * 2 + c

    ebase = (c * 16 + s) * EPT
    cl_ = pltpu.async_copy(col_hbm.at[pl.ds(ebase, EPT)], colbuf, semc)

    erows = [rows0, rows1]
    nfull = XCH // 32
    rem = XCH % 32
    my_chunks = [w + 32 * k for k in range(nfull)]
    for k, ch in enumerate(my_chunks):
        pltpu.sync_copy(x_hbm.at[pl.ds(ch * KE, KE)],
                        xidx.at[pl.ds(k * KE, KE)])
    extra = w + 32 * nfull
    @pl.when(w < rem)
    def _():
        pltpu.sync_copy(x_hbm.at[pl.ds(extra * KE, KE)],
                        xidx.at[pl.ds(nfull * KE, KE)])

    def fire_emb(k, buf):
        return pltpu.async_copy(
            embed_hbm.at[xidx.at[pl.ds(k * KE, KE)]], erows[buf], seme)

    h0_ = fire_emb(0, 0)
    h1_ = fire_emb(1, 1)

    zv = jnp.zeros((16,), jnp.float32)
    ov = jnp.full((16,), 1.0, jnp.float32)
    lane = lax.iota(jnp.int32, 16)
    NR = N // 2

    cl_.wait()

    for p in range(2):
        lo = p * NR

        def zero_it(i, _):
            for l in range(16):
                hist[l, pl.ds(i * 16, 16)] = zv
            return 0
        lax.fori_loop(0, NR // 16, zero_it, 0)

        def deg_it(i, _):
            colv = colbuf[pl.ds(i * 16, 16)]
            il = colv - lo
            msk = (il >= 0) & (il < NR)
            ilc = jnp.clip(il, 0, NR - 1)
            cur = plsc.load_gather(hist, [lane, ilc], mask=msk)
            plsc.store_scatter(hist, [lane, ilc], cur + ov, mask=msk)
            return 0
        lax.fori_loop(0, EPT // 16, deg_it, 0)

        pltpu.sync_copy(hist, degp_hbm.at[p, c, s])

    h0_.wait()
    pltpu.sync_copy(erows[0], h0_hbm.at[pl.ds(my_chunks[0] * KE, KE)])
    h2_ = fire_emb(2, 0)
    h1_.wait()
    pltpu.sync_copy(erows[1], h0_hbm.at[pl.ds(my_chunks[1] * KE, KE)])
    @pl.when(w < rem)
    def _():
        fire_emb(nfull, 1)
    h2_.wait()
    pltpu.sync_copy(erows[0], h0_hbm.at[pl.ds(my_chunks[2] * KE, KE)])
    @pl.when(w < rem)
    def _():
        pltpu.make_async_copy(embed_hbm.at[xidx.at[pl.ds(nfull * KE, KE)]],
                              erows[1], seme).wait()
        pltpu.sync_copy(erows[1], h0_hbm.at[pl.ds(extra * KE, KE)])


@functools.partial(
    pl.kernel,
    out_type=jax.ShapeDtypeStruct((2, N, D), jnp.float32),
    mesh=_mesh,
    scratch_types=[
        pltpu.VMEM_SHARED((N, D), jnp.float32),
        pltpu.VMEM((KE,), jnp.int32),
        pltpu.VMEM((KE,), jnp.int32),
        pltpu.VMEM((KE,), jnp.int32),
        pltpu.VMEM((KE,), jnp.int32),
        pltpu.VMEM((KE,), jnp.int32),
        pltpu.VMEM((KE,), jnp.int32),
        pltpu.VMEM((KE,), jnp.int32),
        pltpu.VMEM((KE,), jnp.int32),
        pltpu.VMEM((KE, D), jnp.float32),
        pltpu.VMEM((KE, D), jnp.float32),
        pltpu.VMEM((KE, D), jnp.float32),
        pltpu.VMEM((KE, D), jnp.float32),
        pltpu.SemaphoreType.DMA,
        pltpu.SemaphoreType.DMA,
        pltpu.SemaphoreType.DMA,
        pltpu.SemaphoreType.DMA,
        pltpu.SemaphoreType.DMA,
        pltpu.SemaphoreType.DMA,
        pltpu.SemaphoreType.DMA,
        pltpu.SemaphoreType.DMA,
        pltpu.SemaphoreType.DMA,
        pltpu.SemaphoreType.DMA,
        pltpu.SemaphoreType.DMA,
        pltpu.SemaphoreType.DMA,
    ],
)
def _sc_edge(row_hbm, col_hbm, g_hbm, accp_hbm,
             acc, ri0, ri1, ri2, ri3, ci0, ci1, ci2, ci3,
             rw0, rw1, rw2, rw3,
             si0, si1, si2, si3, sg0, sg1, sg2, sg3, ss0, ss1, ss2, ss3):
    c = lax.axis_index("c")
    s = lax.axis_index("s")
    ridx = [ri0, ri1, ri2, ri3]
    cidx = [ci0, ci1, ci2, ci3]
    rows = [rw0, rw1, rw2, rw3]
    semi = [si0, si1, si2, si3]
    semg = [sg0, sg1, sg2, sg3]
    sems = [ss0, ss1, ss2, ss3]

    zv = jnp.zeros((16,), jnp.float32)

    def fill(i, _):
        for jj in range(D // 16):
            rw0[i, pl.ds(jj * 16, 16)] = zv
        return 0
    lax.fori_loop(0, KE, fill, 0)

    ebase = (c * 16 + s) * EPT

    def load_idx(j, b):
        off = pl.multiple_of(ebase + j * KE, 8)
        pltpu.async_copy(row_hbm.at[pl.ds(off, KE)], ridx[b], semi[b])
        pltpu.async_copy(col_hbm.at[pl.ds(off, KE)], cidx[b], semi[b])

    def wait_idx(b):
        pltpu.make_async_copy(row_hbm.at[pl.ds(0, KE)], ridx[b], semi[b]).wait()
        pltpu.make_async_copy(col_hbm.at[pl.ds(0, KE)], cidx[b], semi[b]).wait()

    def gather(b):
        pltpu.async_copy(g_hbm.at[ridx[b]], rows[b], semg[b])

    def wait_gather(b):
        pltpu.make_async_copy(g_hbm.at[ridx[b]], rows[b], semg[b]).wait()

    def scatter(b):
        pltpu.async_copy(rows[b], acc.at[cidx[b]], sems[b], add=True)

    def wait_scatter(b):
        pltpu.make_async_copy(rows[b], acc.at[cidx[b]], sems[b]).wait()

    load_idx(0, 0)
    load_idx(1, 1)
    load_idx(2, 2)
    _rr16(s, XCH, lambda ch: pltpu.sync_copy(rw0, acc.at[pl.ds(ch * KE, KE)]))
    plsc.subcore_barrier()

    wait_idx(0)
    gather(0)
    wait_idx(1)
    gather(1)

    wait_gather(0)
    scatter(0)
    load_idx(3, 3)
    wait_idx(2)
    gather(2)

    def edge_it(k, _):
        for i in range(4):
            j = 4 * k + 1 + i
            b = (1 + i) % 4
            wait_gather(b)
            scatter(b)
            wait_scatter((b + 3) % 4)
            load_idx(j + 3, (b + 3) % 4)
            wait_idx((b + 2) % 4)
            gather((b + 2) % 4)
        return 0
    lax.fori_loop(0, 30, edge_it, 0)

    wait_gather(1)
    scatter(1)
    wait_scatter(0)
    load_idx(124, 0)
    wait_idx(3)
    gather(3)
    wait_gather(2)
    scatter(2)
    wait_idx(0)
    gather(0)
    wait_gather(3)
    scatter(3)
    wait_gather(0)
    scatter(0)
    wait_scatter(1)
    wait_scatter(2)
    wait_scatter(3)
    wait_scatter(0)

    plsc.subcore_barrier()
    _rr16(s, XCH, lambda ch: pltpu.sync_copy(
        acc.at[pl.ds(ch * KE, KE)], accp_hbm.at[c, pl.ds(ch * KE, KE)]))


def _tc1_body(degp_ref, h0_ref, w1_ref, g1_ref, dis_ref):
    ones512 = jnp.ones((512, 1), jnp.float32)
    dims = (((0,), (0,)), ((), ()))
    d = jnp.concatenate(
        [lax.dot_general(degp_ref[0], ones512, dims,
                         preferred_element_type=jnp.float32),
         lax.dot_general(degp_ref[1], ones512, dims,
                         preferred_element_type=jnp.float32)], axis=0)
    dis = lax.rsqrt(d + 1.0)
    g1_ref[...] = jnp.dot(h0_ref[...], w1_ref[...],
                          preferred_element_type=jnp.float32) * dis
    dis_ref[...] = dis


def _tc2_body(accp_ref, g1_ref, dis_ref, b1_ref, w2_ref, g2_ref):
    dis = dis_ref[...]
    h1 = jnp.maximum(dis * (accp_ref[0] + accp_ref[1] + g1_ref[...])
                     + b1_ref[...], 0.0)
    g2_ref[...] = jnp.dot(h1, w2_ref[...],
                          preferred_element_type=jnp.float32) * dis


def _tc3_body(accp_ref, g2_ref, dis_ref, b2_ref, batch_ref, wl_ref, bl_ref,
              out_ref):
    dis = dis_ref[...]
    h2 = jnp.maximum(dis * (accp_ref[0] + accp_ref[1] + g2_ref[...])
                     + b2_ref[...], 0.0)
    gid = lax.broadcasted_iota(jnp.int32, (N, NG), 1)
    eh = (batch_ref[...] == gid).astype(jnp.float32)
    sums = lax.dot_general(eh, h2, (((0,), (0,)), ((), ())),
                           preferred_element_type=jnp.float32)
    cnt = lax.dot_general(eh, jnp.ones((N, 1), jnp.float32),
                          (((0,), (0,)), ((), ())),
                          preferred_element_type=jnp.float32)
    pooled = sums / jnp.maximum(cnt, 1.0)
    out_ref[...] = jnp.dot(pooled, wl_ref[...],
                           preferred_element_type=jnp.float32) + bl_ref[...]


_tc1 = pl.pallas_call(
    _tc1_body,
    out_shape=(jax.ShapeDtypeStruct((N, D), jnp.float32),
               jax.ShapeDtypeStruct((N, 1), jnp.float32)))
_tc2 = pl.pallas_call(
    _tc2_body,
    out_shape=jax.ShapeDtypeStruct((N, D), jnp.float32))
_tc3 = pl.pallas_call(
    _tc3_body,
    out_shape=jax.ShapeDtypeStruct((NG, NCLS), jnp.float32))


def kernel(x, edge_index, batch, embed, W1, b1, W2, b2, Wl, bl):
    x = x.reshape(-1).astype(jnp.int32)
    row = edge_index[0].astype(jnp.int32)
    col = edge_index[1].astype(jnp.int32)
    batch2 = batch.reshape(-1, 1).astype(jnp.int32)

    degp, h0 = _sc_deg_emb(col, x, embed)
    g1, dis = _tc1(degp.reshape(2, 512, N // 2), h0, W1)
    acc1 = _sc_edge(row, col, g1)
    g2 = _tc2(acc1, g1, dis, b1.reshape(1, -1), W2)
    acc2 = _sc_edge(row, col, g2)
    return _tc3(acc2, g2, dis, b2.reshape(1, -1), batch2, Wl, bl.reshape(1, -1))
